# initial kernel scaffold (unmeasured)
import jax
import jax.numpy as jnp
from jax import lax
from jax.experimental import pallas as pl
from jax.experimental.pallas import tpu as pltpu

N_Z = 4


def kernel(x, dy):
    k, m = x.shape
    k2, f = dy.shape
    chunk = m // N_Z
    half = f // 2

    x_bf = x.astype(jnp.bfloat16)
    dy_bf = dy.astype(jnp.bfloat16)

    def body(x_ref, dy_ref, out_ref, partial_ref, recv_p, recv_m,
             send_sems, recv_sems):
        my_x = lax.axis_index("x")
        my_y = lax.axis_index("y")
        my_z = lax.axis_index("z")
        up = (my_z + 1) % N_Z
        dn = (my_z - 1) % N_Z

        barrier_sem = pltpu.get_barrier_semaphore()
        for nbr in (up, dn):
            pl.semaphore_signal(
                barrier_sem, inc=1,
                device_id=(my_x, my_y, nbr),
                device_id_type=pl.DeviceIdType.MESH,
            )
        pl.semaphore_wait(barrier_sem, 2)

        partial_ref[...] = lax.dot_general(
            x_ref[...], dy_ref[...],
            dimension_numbers=(((0,), (0,)), ((), ())),
            preferred_element_type=jnp.bfloat16,
        )

        for s in range(N_Z - 1):
            send_c_p = (my_z - s - 1) % N_Z
            recv_c_p = (my_z - s - 2) % N_Z
            send_c_m = (my_z + s + 1) % N_Z
            recv_c_m = (my_z + s + 2) % N_Z

            rdma_p = pltpu.make_async_remote_copy(
                src_ref=partial_ref.at[pl.ds(send_c_p * chunk, chunk),
                                       pl.ds(0, half)],
                dst_ref=recv_p.at[s],
                send_sem=send_sems.at[0, s],
                recv_sem=recv_sems.at[0, s],
                device_id=(my_x, my_y, up),
                device_id_type=pl.DeviceIdType.MESH,
            )
            rdma_m = pltpu.make_async_remote_copy(
                src_ref=partial_ref.at[pl.ds(send_c_m * chunk, chunk),
                                       pl.ds(half, half)],
                dst_ref=recv_m.at[s],
                send_sem=send_sems.at[1, s],
                recv_sem=recv_sems.at[1, s],
                device_id=(my_x, my_y, dn),
                device_id_type=pl.DeviceIdType.MESH,
            )
            rdma_p.start()
            rdma_m.start()
            rdma_p.wait()
            rdma_m.wait()

            partial_ref[pl.ds(recv_c_p * chunk, chunk), pl.ds(0, half)] = (
                partial_ref[pl.ds(recv_c_p * chunk, chunk), pl.ds(0, half)]
                + recv_p[s]
            )
            partial_ref[pl.ds(recv_c_m * chunk, chunk), pl.ds(half, half)] = (
                partial_ref[pl.ds(recv_c_m * chunk, chunk), pl.ds(half, half)]
                + recv_m[s]
            )

        out_ref[...] = partial_ref[pl.ds(my_z * chunk, chunk), :].astype(
            jnp.float32
        )

    return pl.pallas_call(
        body,
        out_shape=jax.ShapeDtypeStruct((chunk, f), jnp.float32),
        in_specs=[
            pl.BlockSpec(memory_space=pltpu.VMEM),
            pl.BlockSpec(memory_space=pltpu.VMEM),
        ],
        out_specs=pl.BlockSpec(memory_space=pltpu.VMEM),
        scratch_shapes=[
            pltpu.VMEM((m, f), jnp.bfloat16),
            pltpu.VMEM((N_Z - 1, chunk, half), jnp.bfloat16),
            pltpu.VMEM((N_Z - 1, chunk, half), jnp.bfloat16),
            pltpu.SemaphoreType.DMA((2, N_Z - 1)),
            pltpu.SemaphoreType.DMA((2, N_Z - 1)),
        ],
        compiler_params=pltpu.CompilerParams(
            collective_id=0,
            vmem_limit_bytes=128 * 1024 * 1024,
        ),
    )(x_bf, dy_bf)


# baseline (device time: 186821 ns/iter reference)
import jax
import jax.numpy as jnp
from jax import lax
from jax.experimental import pallas as pl
from jax.experimental.pallas import tpu as pltpu

N_Z = 4


def kernel(x, dy):
    k, m = x.shape
    _, f = dy.shape
    chunk = m // N_Z
    quarter = f // 4

    qx = lax.axis_index("x")
    qy = lax.axis_index("y")
    q_outer = 2 * qx + qy
    x_bf = x.astype(jnp.bfloat16)
    dy_q = lax.dynamic_slice(dy, (0, q_outer * quarter), (k, quarter))
    dy_bf = dy_q.astype(jnp.bfloat16)

    def body(x_ref, dy_ref, out_ref, partial_ref, up_recv, dn_recv,
             quarters_ref, stage_ref, z_send_sems, z_recv_sems,
             xy_send_sems, xy_recv_sems, stage_sems, local_sem):
        my_x = lax.axis_index("x")
        my_y = lax.axis_index("y")
        my_z = lax.axis_index("z")
        q = 2 * my_x + my_y
        has_up = my_z < N_Z - 1
        has_dn = my_z > 0

        barrier_sem = pltpu.get_barrier_semaphore()

        @pl.when(has_up)
        def _():
            pl.semaphore_signal(
                barrier_sem, inc=1, device_id=(my_x, my_y, my_z + 1),
                device_id_type=pl.DeviceIdType.MESH)

        @pl.when(has_dn)
        def _():
            pl.semaphore_signal(
                barrier_sem, inc=1, device_id=(my_x, my_y, my_z - 1),
                device_id_type=pl.DeviceIdType.MESH)

        pl.semaphore_signal(
            barrier_sem, inc=1, device_id=(1 - my_x, my_y, my_z),
            device_id_type=pl.DeviceIdType.MESH)
        pl.semaphore_signal(
            barrier_sem, inc=1, device_id=(my_x, 1 - my_y, my_z),
            device_id_type=pl.DeviceIdType.MESH)
        n_nbrs = 2 + has_up.astype(jnp.int32) + has_dn.astype(jnp.int32)
        pl.semaphore_wait(barrier_sem, n_nbrs)

        for r in range(N_Z):
            partial_ref[r * chunk:(r + 1) * chunk, :] = lax.dot_general(
                x_ref[:, r * chunk:(r + 1) * chunk], dy_ref[...],
                dimension_numbers=(((0,), (0,)), ((), ())),
                preferred_element_type=jnp.float32,
            ).astype(jnp.bfloat16)

        for s in range(N_Z - 1):
            up_c = my_z + (N_Z - 1) - s
            dn_c = my_z - (N_Z - 1) + s
            up_send_ok = jnp.logical_and(has_up, s >= my_z)
            dn_send_ok = jnp.logical_and(has_dn, s >= (N_Z - 1) - my_z)
            up_recv_ok = jnp.logical_and(has_dn, s >= my_z - 1)
            dn_recv_ok = jnp.logical_and(has_up, s >= (N_Z - 2) - my_z)

            @pl.when(up_send_ok)
            def _():
                pltpu.make_async_remote_copy(
                    src_ref=partial_ref.at[pl.ds(up_c * chunk, chunk), :],
                    dst_ref=up_recv.at[s],
                    send_sem=z_send_sems.at[0, s],
                    recv_sem=z_recv_sems.at[0, s],
                    device_id=(my_x, my_y, my_z + 1),
                    device_id_type=pl.DeviceIdType.MESH,
                ).start()

            @pl.when(dn_send_ok)
            def _():
                pltpu.make_async_remote_copy(
                    src_ref=partial_ref.at[pl.ds(dn_c * chunk, chunk), :],
                    dst_ref=dn_recv.at[s],
                    send_sem=z_send_sems.at[1, s],
                    recv_sem=z_recv_sems.at[1, s],
                    device_id=(my_x, my_y, my_z - 1),
                    device_id_type=pl.DeviceIdType.MESH,
                ).start()

            @pl.when(up_recv_ok)
            def _():
                pltpu.make_async_remote_copy(
                    src_ref=up_recv.at[s], dst_ref=up_recv.at[s],
                    send_sem=z_send_sems.at[0, s],
                    recv_sem=z_recv_sems.at[0, s],
                    device_id=(my_x, my_y, my_z),
                    device_id_type=pl.DeviceIdType.MESH,
                ).wait_recv()
                c = my_z + (N_Z - 2) - s
                partial_ref[pl.ds(c * chunk, chunk), :] = (
                    partial_ref[pl.ds(c * chunk, chunk), :] + up_recv[s]
                )

            @pl.when(dn_recv_ok)
            def _():
                pltpu.make_async_remote_copy(
                    src_ref=dn_recv.at[s], dst_ref=dn_recv.at[s],
                    send_sem=z_send_sems.at[1, s],
                    recv_sem=z_recv_sems.at[1, s],
                    device_id=(my_x, my_y, my_z),
                    device_id_type=pl.DeviceIdType.MESH,
                ).wait_recv()
                c = my_z - (N_Z - 2) + s
                partial_ref[pl.ds(c * chunk, chunk), :] = (
                    partial_ref[pl.ds(c * chunk, chunk), :] + dn_recv[s]
                )

            @pl.when(up_send_ok)
            def _():
                pltpu.make_async_remote_copy(
                    src_ref=partial_ref.at[pl.ds(up_c * chunk, chunk), :],
                    dst_ref=up_recv.at[s],
                    send_sem=z_send_sems.at[0, s],
                    recv_sem=z_recv_sems.at[0, s],
                    device_id=(my_x, my_y, my_z),
                    device_id_type=pl.DeviceIdType.MESH,
                ).wait_send()

            @pl.when(dn_send_ok)
            def _():
                pltpu.make_async_remote_copy(
                    src_ref=partial_ref.at[pl.ds(dn_c * chunk, chunk), :],
                    dst_ref=dn_recv.at[s],
                    send_sem=z_send_sems.at[1, s],
                    recv_sem=z_recv_sems.at[1, s],
                    device_id=(my_x, my_y, my_z),
                    device_id_type=pl.DeviceIdType.MESH,
                ).wait_send()

        own_copy = pltpu.make_async_copy(
            partial_ref.at[pl.ds(my_z * chunk, chunk), :],
            quarters_ref.at[q],
            local_sem,
        )
        own_copy.start()
        own_copy.wait()

        x_ex = pltpu.make_async_remote_copy(
            src_ref=quarters_ref.at[q], dst_ref=quarters_ref.at[q],
            send_sem=xy_send_sems.at[0], recv_sem=xy_recv_sems.at[0],
            device_id=(1 - my_x, my_y, my_z),
            device_id_type=pl.DeviceIdType.MESH,
        )
        y_ex1 = pltpu.make_async_remote_copy(
            src_ref=quarters_ref.at[q], dst_ref=quarters_ref.at[q],
            send_sem=xy_send_sems.at[1], recv_sem=xy_recv_sems.at[1],
            device_id=(my_x, 1 - my_y, my_z),
            device_id_type=pl.DeviceIdType.MESH,
        )
        x_ex.start()
        y_ex1.start()
        x_ex.wait()
        q2 = q ^ 2
        y_ex2 = pltpu.make_async_remote_copy(
            src_ref=quarters_ref.at[q2], dst_ref=quarters_ref.at[q2],
            send_sem=xy_send_sems.at[2], recv_sem=xy_recv_sems.at[2],
            device_id=(my_x, 1 - my_y, my_z),
            device_id_type=pl.DeviceIdType.MESH,
        )
        y_ex2.start()
        y_ex1.wait()
        y_ex2.wait()

        def out_copy(j, slot):
            return pltpu.make_async_copy(
                stage_ref.at[slot],
                out_ref.at[:, j * quarter:(j + 1) * quarter],
                stage_sems.at[slot],
            )

        for j in range(4):
            slot = j % 2
            if j >= 2:
                out_copy(j - 2, slot).wait()
            stage_ref[slot] = quarters_ref[j].astype(jnp.float32)
            out_copy(j, slot).start()
        out_copy(2, 0).wait()
        out_copy(3, 1).wait()

    return pl.pallas_call(
        body,
        out_shape=jax.ShapeDtypeStruct((chunk, f), jnp.float32),
        in_specs=[
            pl.BlockSpec(memory_space=pltpu.VMEM),
            pl.BlockSpec(memory_space=pltpu.VMEM),
        ],
        out_specs=pl.BlockSpec(memory_space=pl.ANY),
        scratch_shapes=[
            pltpu.VMEM((m, quarter), jnp.bfloat16),
            pltpu.VMEM((N_Z - 1, chunk, quarter), jnp.bfloat16),
            pltpu.VMEM((N_Z - 1, chunk, quarter), jnp.bfloat16),
            pltpu.VMEM((4, chunk, quarter), jnp.bfloat16),
            pltpu.VMEM((2, chunk, quarter), jnp.float32),
            pltpu.SemaphoreType.DMA((2, N_Z - 1)),
            pltpu.SemaphoreType.DMA((2, N_Z - 1)),
            pltpu.SemaphoreType.DMA((3,)),
            pltpu.SemaphoreType.DMA((3,)),
            pltpu.SemaphoreType.DMA((2,)),
            pltpu.SemaphoreType.DMA,
        ],
        compiler_params=pltpu.CompilerParams(
            collective_id=0,
            vmem_limit_bytes=64 * 1024 * 1024,
        ),
    )(x_bf, dy_bf)


# device time: 173864 ns/iter; 1.0745x vs baseline; 1.0745x over previous
import jax
import jax.numpy as jnp
from jax import lax
from jax.experimental import pallas as pl
from jax.experimental.pallas import tpu as pltpu

N_Z = 4


def kernel(x, dy):
    k, m = x.shape
    _, f = dy.shape
    chunk = m // N_Z
    quarter = f // 4

    qx = lax.axis_index("x")
    qy = lax.axis_index("y")
    q_outer = 2 * qx + qy
    x_bf = x.astype(jnp.bfloat16)
    dy_q = lax.dynamic_slice(dy, (0, q_outer * quarter), (k, quarter))
    dy_bf = dy_q.astype(jnp.bfloat16)

    def body(x_ref, dy_ref, out_ref, partial_ref, up_recv, dn_recv,
             x_recv, y1_recv, y2_recv, stage_ref, z_send_sems,
             z_recv_sems, xy_send_sems, xy_recv_sems, stage_sems):
        my_x = lax.axis_index("x")
        my_y = lax.axis_index("y")
        my_z = lax.axis_index("z")
        q = 2 * my_x + my_y
        has_up = my_z < N_Z - 1
        has_dn = my_z > 0

        def up_send_ok(s):
            return jnp.logical_and(has_up, s >= my_z)

        def dn_send_ok(s):
            return jnp.logical_and(has_dn, s >= (N_Z - 1) - my_z)

        def start_z_sends(s):
            up_c = my_z + (N_Z - 1) - s
            dn_c = my_z - (N_Z - 1) + s

            @pl.when(up_send_ok(s))
            def _():
                pltpu.make_async_remote_copy(
                    src_ref=partial_ref.at[pl.ds(up_c * chunk, chunk), :],
                    dst_ref=up_recv.at[s],
                    send_sem=z_send_sems.at[0, s],
                    recv_sem=z_recv_sems.at[0, s],
                    device_id=(my_x, my_y, my_z + 1),
                    device_id_type=pl.DeviceIdType.MESH,
                ).start()

            @pl.when(dn_send_ok(s))
            def _():
                pltpu.make_async_remote_copy(
                    src_ref=partial_ref.at[pl.ds(dn_c * chunk, chunk), :],
                    dst_ref=dn_recv.at[s],
                    send_sem=z_send_sems.at[1, s],
                    recv_sem=z_recv_sems.at[1, s],
                    device_id=(my_x, my_y, my_z - 1),
                    device_id_type=pl.DeviceIdType.MESH,
                ).start()

        barrier_sem = pltpu.get_barrier_semaphore()

        @pl.when(has_up)
        def _():
            pl.semaphore_signal(
                barrier_sem, inc=1, device_id=(my_x, my_y, my_z + 1),
                device_id_type=pl.DeviceIdType.MESH)

        @pl.when(has_dn)
        def _():
            pl.semaphore_signal(
                barrier_sem, inc=1, device_id=(my_x, my_y, my_z - 1),
                device_id_type=pl.DeviceIdType.MESH)

        pl.semaphore_signal(
            barrier_sem, inc=1, device_id=(1 - my_x, my_y, my_z),
            device_id_type=pl.DeviceIdType.MESH)
        pl.semaphore_signal(
            barrier_sem, inc=1, device_id=(my_x, 1 - my_y, my_z),
            device_id_type=pl.DeviceIdType.MESH)
        n_nbrs = 2 + has_up.astype(jnp.int32) + has_dn.astype(jnp.int32)
        pl.semaphore_wait(barrier_sem, n_nbrs)

        def gemm_block(r):
            partial_ref[r * chunk:(r + 1) * chunk, :] = lax.dot_general(
                x_ref[:, r * chunk:(r + 1) * chunk], dy_ref[...],
                dimension_numbers=(((0,), (0,)), ((), ())),
                preferred_element_type=jnp.float32,
            ).astype(jnp.bfloat16)

        gemm_block(N_Z - 1)
        gemm_block(0)
        start_z_sends(0)
        gemm_block(2)
        gemm_block(1)

        for s in range(N_Z - 1):
            if s > 0:
                start_z_sends(s)

            @pl.when(jnp.logical_and(has_dn, s >= my_z - 1))
            def _():
                pltpu.make_async_remote_copy(
                    src_ref=up_recv.at[s], dst_ref=up_recv.at[s],
                    send_sem=z_send_sems.at[0, s],
                    recv_sem=z_recv_sems.at[0, s],
                    device_id=(my_x, my_y, my_z),
                    device_id_type=pl.DeviceIdType.MESH,
                ).wait_recv()
                c = my_z + (N_Z - 2) - s
                partial_ref[pl.ds(c * chunk, chunk), :] = (
                    partial_ref[pl.ds(c * chunk, chunk), :] + up_recv[s]
                )

            @pl.when(jnp.logical_and(has_up, s >= (N_Z - 2) - my_z))
            def _():
                pltpu.make_async_remote_copy(
                    src_ref=dn_recv.at[s], dst_ref=dn_recv.at[s],
                    send_sem=z_send_sems.at[1, s],
                    recv_sem=z_recv_sems.at[1, s],
                    device_id=(my_x, my_y, my_z),
                    device_id_type=pl.DeviceIdType.MESH,
                ).wait_recv()
                c = my_z - (N_Z - 2) + s
                partial_ref[pl.ds(c * chunk, chunk), :] = (
                    partial_ref[pl.ds(c * chunk, chunk), :] + dn_recv[s]
                )

        own_rows = partial_ref.at[pl.ds(my_z * chunk, chunk), :]
        x_ex = pltpu.make_async_remote_copy(
            src_ref=own_rows, dst_ref=x_recv,
            send_sem=xy_send_sems.at[0], recv_sem=xy_recv_sems.at[0],
            device_id=(1 - my_x, my_y, my_z),
            device_id_type=pl.DeviceIdType.MESH,
        )
        y_ex1 = pltpu.make_async_remote_copy(
            src_ref=own_rows, dst_ref=y1_recv,
            send_sem=xy_send_sems.at[1], recv_sem=xy_recv_sems.at[1],
            device_id=(my_x, 1 - my_y, my_z),
            device_id_type=pl.DeviceIdType.MESH,
        )
        x_ex.start()
        y_ex1.start()

        def out_copy(col_q, slot):
            return pltpu.make_async_copy(
                stage_ref.at[slot],
                out_ref.at[:, pl.ds(col_q * quarter, quarter)],
                stage_sems.at[slot],
            )

        stage_ref[0] = partial_ref[pl.ds(my_z * chunk, chunk), :].astype(
            jnp.float32)
        out_copy(q, 0).start()

        x_ex.wait()
        y_ex2 = pltpu.make_async_remote_copy(
            src_ref=x_recv, dst_ref=y2_recv,
            send_sem=xy_send_sems.at[2], recv_sem=xy_recv_sems.at[2],
            device_id=(my_x, 1 - my_y, my_z),
            device_id_type=pl.DeviceIdType.MESH,
        )
        y_ex2.start()
        stage_ref[1] = x_recv[...].astype(jnp.float32)
        out_copy(q ^ 2, 1).start()

        y_ex1.wait()
        out_copy(q, 0).wait()
        stage_ref[0] = y1_recv[...].astype(jnp.float32)
        out_copy(q ^ 1, 0).start()

        y_ex2.wait()
        out_copy(q ^ 2, 1).wait()
        stage_ref[1] = y2_recv[...].astype(jnp.float32)
        out_copy(q ^ 3, 1).start()

        out_copy(q ^ 1, 0).wait()
        out_copy(q ^ 3, 1).wait()

        for s in range(N_Z - 1):
            up_c = my_z + (N_Z - 1) - s
            dn_c = my_z - (N_Z - 1) + s

            @pl.when(up_send_ok(s))
            def _():
                pltpu.make_async_remote_copy(
                    src_ref=partial_ref.at[pl.ds(up_c * chunk, chunk), :],
                    dst_ref=up_recv.at[s],
                    send_sem=z_send_sems.at[0, s],
                    recv_sem=z_recv_sems.at[0, s],
                    device_id=(my_x, my_y, my_z),
                    device_id_type=pl.DeviceIdType.MESH,
                ).wait_send()

            @pl.when(dn_send_ok(s))
            def _():
                pltpu.make_async_remote_copy(
                    src_ref=partial_ref.at[pl.ds(dn_c * chunk, chunk), :],
                    dst_ref=dn_recv.at[s],
                    send_sem=z_send_sems.at[1, s],
                    recv_sem=z_recv_sems.at[1, s],
                    device_id=(my_x, my_y, my_z),
                    device_id_type=pl.DeviceIdType.MESH,
                ).wait_send()

    return pl.pallas_call(
        body,
        out_shape=jax.ShapeDtypeStruct((chunk, f), jnp.float32),
        in_specs=[
            pl.BlockSpec(memory_space=pltpu.VMEM),
            pl.BlockSpec(memory_space=pltpu.VMEM),
        ],
        out_specs=pl.BlockSpec(memory_space=pl.ANY),
        scratch_shapes=[
            pltpu.VMEM((m, quarter), jnp.bfloat16),
            pltpu.VMEM((N_Z - 1, chunk, quarter), jnp.bfloat16),
            pltpu.VMEM((N_Z - 1, chunk, quarter), jnp.bfloat16),
            pltpu.VMEM((chunk, quarter), jnp.bfloat16),
            pltpu.VMEM((chunk, quarter), jnp.bfloat16),
            pltpu.VMEM((chunk, quarter), jnp.bfloat16),
            pltpu.VMEM((2, chunk, quarter), jnp.float32),
            pltpu.SemaphoreType.DMA((2, N_Z - 1)),
            pltpu.SemaphoreType.DMA((2, N_Z - 1)),
            pltpu.SemaphoreType.DMA((3,)),
            pltpu.SemaphoreType.DMA((3,)),
            pltpu.SemaphoreType.DMA((2,)),
        ],
        compiler_params=pltpu.CompilerParams(
            collective_id=0,
            vmem_limit_bytes=64 * 1024 * 1024,
        ),
    )(x_bf, dy_bf)


# device time: 145798 ns/iter; 1.2814x vs baseline; 1.1925x over previous
import jax
import jax.numpy as jnp
from jax import lax
from jax.experimental import pallas as pl
from jax.experimental.pallas import tpu as pltpu

N_Z = 4
N_H = 2
N_S = N_Z - 1


def kernel(x, dy):
    k, m = x.shape
    _, f = dy.shape
    chunk = m // N_Z
    quarter = f // 4
    halfq = quarter // N_H

    qx = lax.axis_index("x")
    qy = lax.axis_index("y")
    q_outer = 2 * qx + qy
    x_bf = x.astype(jnp.bfloat16)
    dy_q = lax.dynamic_slice(dy, (0, q_outer * quarter), (k, quarter))
    dy_bf = dy_q.astype(jnp.bfloat16)

    def body(x_ref, dy_ref, out_ref, partial_ref, up_recv, dn_recv,
             x_recv, y1_recv, y2_recv, stage_ref, z_send_sems,
             z_recv_sems, xy_send_sems, xy_recv_sems, stage_sems):
        my_x = lax.axis_index("x")
        my_y = lax.axis_index("y")
        my_z = lax.axis_index("z")
        q = 2 * my_x + my_y
        has_up = my_z < N_Z - 1
        has_dn = my_z > 0

        def up_send_ok(s):
            return jnp.logical_and(has_up, s >= my_z)

        def dn_send_ok(s):
            return jnp.logical_and(has_dn, s >= (N_Z - 1) - my_z)

        def rows(c):
            return pl.ds(c * chunk, chunk)

        def hcols(h):
            return pl.ds(h * halfq, halfq)

        def z_rdma(dir_i, s, h, c, target_z):
            recv = up_recv if dir_i == 0 else dn_recv
            return pltpu.make_async_remote_copy(
                src_ref=partial_ref.at[rows(c), hcols(h)],
                dst_ref=recv.at[s, h],
                send_sem=z_send_sems.at[dir_i, s, h],
                recv_sem=z_recv_sems.at[dir_i, s, h],
                device_id=(my_x, my_y, target_z),
                device_id_type=pl.DeviceIdType.MESH,
            )

        def start_z_sends(s, h):
            up_c = my_z + (N_Z - 1) - s
            dn_c = my_z - (N_Z - 1) + s

            @pl.when(up_send_ok(s))
            def _():
                z_rdma(0, s, h, up_c, my_z + 1).start()

            @pl.when(dn_send_ok(s))
            def _():
                z_rdma(1, s, h, dn_c, my_z - 1).start()

        def recv_and_acc(s, h):
            @pl.when(jnp.logical_and(has_dn, s >= my_z - 1))
            def _():
                z_rdma(0, s, h, 0, my_z).wait_recv()
                c = my_z + (N_Z - 2) - s
                partial_ref[rows(c), hcols(h)] = (
                    partial_ref[rows(c), hcols(h)] + up_recv[s, h]
                )

            @pl.when(jnp.logical_and(has_up, s >= (N_Z - 2) - my_z))
            def _():
                z_rdma(1, s, h, 0, my_z).wait_recv()
                c = my_z - (N_Z - 2) + s
                partial_ref[rows(c), hcols(h)] = (
                    partial_ref[rows(c), hcols(h)] + dn_recv[s, h]
                )

        barrier_sem = pltpu.get_barrier_semaphore()

        @pl.when(has_up)
        def _():
            pl.semaphore_signal(
                barrier_sem, inc=1, device_id=(my_x, my_y, my_z + 1),
                device_id_type=pl.DeviceIdType.MESH)

        @pl.when(has_dn)
        def _():
            pl.semaphore_signal(
                barrier_sem, inc=1, device_id=(my_x, my_y, my_z - 1),
                device_id_type=pl.DeviceIdType.MESH)

        pl.semaphore_signal(
            barrier_sem, inc=1, device_id=(1 - my_x, my_y, my_z),
            device_id_type=pl.DeviceIdType.MESH)
        pl.semaphore_signal(
            barrier_sem, inc=1, device_id=(my_x, 1 - my_y, my_z),
            device_id_type=pl.DeviceIdType.MESH)
        n_nbrs = 2 + has_up.astype(jnp.int32) + has_dn.astype(jnp.int32)
        pl.semaphore_wait(barrier_sem, n_nbrs)

        def gemm_block(r):
            partial_ref[r * chunk:(r + 1) * chunk, :] = lax.dot_general(
                x_ref[:, r * chunk:(r + 1) * chunk], dy_ref[...],
                dimension_numbers=(((0,), (0,)), ((), ())),
                preferred_element_type=jnp.float32,
            ).astype(jnp.bfloat16)

        gemm_block(N_Z - 1)
        gemm_block(0)
        for h in range(N_H):
            start_z_sends(0, h)
        gemm_block(2)
        gemm_block(1)

        own_src = [partial_ref.at[rows(my_z), hcols(h)] for h in range(N_H)]

        def xy_rdma(ex_i, h, src, dst, dev):
            return pltpu.make_async_remote_copy(
                src_ref=src, dst_ref=dst,
                send_sem=xy_send_sems.at[ex_i, h],
                recv_sem=xy_recv_sems.at[ex_i, h],
                device_id=dev, device_id_type=pl.DeviceIdType.MESH,
            )

        x_dev = (1 - my_x, my_y, my_z)
        y_dev = (my_x, 1 - my_y, my_z)

        def x_ex(h):
            return xy_rdma(0, h, own_src[h], x_recv.at[h], x_dev)

        def y_ex1(h):
            return xy_rdma(1, h, own_src[h], y1_recv.at[h], y_dev)

        def y_ex2(h):
            return xy_rdma(2, h, x_recv.at[h], y2_recv.at[h], y_dev)

        drain_state = {"n": 0}

        def out_copy(col_q, h, slot):
            return pltpu.make_async_copy(
                stage_ref.at[slot],
                out_ref.at[:, pl.ds(col_q * quarter + h * halfq, halfq)],
                stage_sems.at[slot],
            )

        pending = [None, None]

        def drain(value, col_q, h):
            slot = drain_state["n"] % 2
            drain_state["n"] += 1
            if pending[slot] is not None:
                out_copy(*pending[slot], slot).wait()
            stage_ref[slot] = value.astype(jnp.float32)
            out_copy(col_q, h, slot).start()
            pending[slot] = (col_q, h)

        for s in range(N_S):
            for h in range(N_H):
                recv_and_acc(s, h)
                if s + 1 < N_S:
                    start_z_sends(s + 1, h)
                else:
                    x_ex(h).start()
                    y_ex1(h).start()

        for h in range(N_H):
            drain(partial_ref[rows(my_z), hcols(h)], q, h)

        for h in range(N_H):
            x_ex(h).wait()
            y_ex2(h).start()
            drain(x_recv[h], q ^ 2, h)

        for h in range(N_H):
            y_ex1(h).wait()
            drain(y1_recv[h], q ^ 1, h)

        for h in range(N_H):
            y_ex2(h).wait()
            drain(y2_recv[h], q ^ 3, h)

        for slot in range(2):
            if pending[slot] is not None:
                out_copy(*pending[slot], slot).wait()

        for s in range(N_S):
            up_c = my_z + (N_Z - 1) - s
            dn_c = my_z - (N_Z - 1) + s
            for h in range(N_H):
                @pl.when(up_send_ok(s))
                def _():
                    z_rdma(0, s, h, up_c, my_z).wait_send()

                @pl.when(dn_send_ok(s))
                def _():
                    z_rdma(1, s, h, dn_c, my_z).wait_send()

    return pl.pallas_call(
        body,
        out_shape=jax.ShapeDtypeStruct((chunk, f), jnp.float32),
        in_specs=[
            pl.BlockSpec(memory_space=pltpu.VMEM),
            pl.BlockSpec(memory_space=pltpu.VMEM),
        ],
        out_specs=pl.BlockSpec(memory_space=pl.ANY),
        scratch_shapes=[
            pltpu.VMEM((m, quarter), jnp.bfloat16),
            pltpu.VMEM((N_S, N_H, chunk, halfq), jnp.bfloat16),
            pltpu.VMEM((N_S, N_H, chunk, halfq), jnp.bfloat16),
            pltpu.VMEM((N_H, chunk, halfq), jnp.bfloat16),
            pltpu.VMEM((N_H, chunk, halfq), jnp.bfloat16),
            pltpu.VMEM((N_H, chunk, halfq), jnp.bfloat16),
            pltpu.VMEM((2, chunk, halfq), jnp.float32),
            pltpu.SemaphoreType.DMA((2, N_S, N_H)),
            pltpu.SemaphoreType.DMA((2, N_S, N_H)),
            pltpu.SemaphoreType.DMA((3, N_H)),
            pltpu.SemaphoreType.DMA((3, N_H)),
            pltpu.SemaphoreType.DMA((2,)),
        ],
        compiler_params=pltpu.CompilerParams(
            collective_id=0,
            vmem_limit_bytes=64 * 1024 * 1024,
        ),
    )(x_bf, dy_bf)


# device time: 134625 ns/iter; 1.3877x vs baseline; 1.0830x over previous
import jax
import jax.numpy as jnp
from jax import lax
from jax.experimental import pallas as pl
from jax.experimental.pallas import tpu as pltpu

N_Z = 4
N_H = 4
N_S = N_Z - 1


def kernel(x, dy):
    k, m = x.shape
    _, f = dy.shape
    chunk = m // N_Z
    quarter = f // 4
    halfq = quarter // N_H

    qx = lax.axis_index("x")
    qy = lax.axis_index("y")
    q_outer = 2 * qx + qy
    x_bf = x.astype(jnp.bfloat16)
    dy_q = lax.dynamic_slice(dy, (0, q_outer * quarter), (k, quarter))
    dy_bf = dy_q.astype(jnp.bfloat16)

    def body(x_ref, dy_ref, out_ref, partial_ref, up_recv, dn_recv,
             x_recv, y1_recv, y2_recv, stage_ref, z_send_sems,
             z_recv_sems, xy_send_sems, xy_recv_sems, stage_sems):
        my_x = lax.axis_index("x")
        my_y = lax.axis_index("y")
        my_z = lax.axis_index("z")
        q = 2 * my_x + my_y
        has_up = my_z < N_Z - 1
        has_dn = my_z > 0

        def up_send_ok(s):
            return jnp.logical_and(has_up, s >= my_z)

        def dn_send_ok(s):
            return jnp.logical_and(has_dn, s >= (N_Z - 1) - my_z)

        def rows(c):
            return pl.ds(c * chunk, chunk)

        def hcols(h):
            return pl.ds(h * halfq, halfq)

        def z_rdma(dir_i, s, h, c, target_z):
            recv = up_recv if dir_i == 0 else dn_recv
            return pltpu.make_async_remote_copy(
                src_ref=partial_ref.at[rows(c), hcols(h)],
                dst_ref=recv.at[s, h],
                send_sem=z_send_sems.at[dir_i, s, h],
                recv_sem=z_recv_sems.at[dir_i, s, h],
                device_id=(my_x, my_y, target_z),
                device_id_type=pl.DeviceIdType.MESH,
            )

        def start_z_sends(s, h):
            up_c = my_z + (N_Z - 1) - s
            dn_c = my_z - (N_Z - 1) + s

            @pl.when(up_send_ok(s))
            def _():
                z_rdma(0, s, h, up_c, my_z + 1).start()

            @pl.when(dn_send_ok(s))
            def _():
                z_rdma(1, s, h, dn_c, my_z - 1).start()

        def recv_and_acc(s, h):
            @pl.when(jnp.logical_and(has_dn, s >= my_z - 1))
            def _():
                z_rdma(0, s, h, 0, my_z).wait_recv()
                c = my_z + (N_Z - 2) - s
                partial_ref[rows(c), hcols(h)] = (
                    partial_ref[rows(c), hcols(h)] + up_recv[s, h]
                )

            @pl.when(jnp.logical_and(has_up, s >= (N_Z - 2) - my_z))
            def _():
                z_rdma(1, s, h, 0, my_z).wait_recv()
                c = my_z - (N_Z - 2) + s
                partial_ref[rows(c), hcols(h)] = (
                    partial_ref[rows(c), hcols(h)] + dn_recv[s, h]
                )

        barrier_sem = pltpu.get_barrier_semaphore()

        @pl.when(has_up)
        def _():
            pl.semaphore_signal(
                barrier_sem, inc=1, device_id=(my_x, my_y, my_z + 1),
                device_id_type=pl.DeviceIdType.MESH)

        @pl.when(has_dn)
        def _():
            pl.semaphore_signal(
                barrier_sem, inc=1, device_id=(my_x, my_y, my_z - 1),
                device_id_type=pl.DeviceIdType.MESH)

        pl.semaphore_signal(
            barrier_sem, inc=1, device_id=(1 - my_x, my_y, my_z),
            device_id_type=pl.DeviceIdType.MESH)
        pl.semaphore_signal(
            barrier_sem, inc=1, device_id=(my_x, 1 - my_y, my_z),
            device_id_type=pl.DeviceIdType.MESH)
        n_nbrs = 2 + has_up.astype(jnp.int32) + has_dn.astype(jnp.int32)
        pl.semaphore_wait(barrier_sem, n_nbrs)

        def gemm_block(r):
            partial_ref[r * chunk:(r + 1) * chunk, :] = lax.dot_general(
                x_ref[:, r * chunk:(r + 1) * chunk], dy_ref[...],
                dimension_numbers=(((0,), (0,)), ((), ())),
                preferred_element_type=jnp.float32,
            ).astype(jnp.bfloat16)

        gemm_block(N_Z - 1)
        gemm_block(0)
        for h in range(N_H):
            start_z_sends(0, h)
        gemm_block(2)
        gemm_block(1)

        own_src = [partial_ref.at[rows(my_z), hcols(h)] for h in range(N_H)]

        def xy_rdma(ex_i, h, src, dst, dev):
            return pltpu.make_async_remote_copy(
                src_ref=src, dst_ref=dst,
                send_sem=xy_send_sems.at[ex_i, h],
                recv_sem=xy_recv_sems.at[ex_i, h],
                device_id=dev, device_id_type=pl.DeviceIdType.MESH,
            )

        x_dev = (1 - my_x, my_y, my_z)
        y_dev = (my_x, 1 - my_y, my_z)

        def x_ex(h):
            return xy_rdma(0, h, own_src[h], x_recv.at[h], x_dev)

        def y_ex1(h):
            return xy_rdma(1, h, own_src[h], y1_recv.at[h], y_dev)

        def y_ex2(h):
            return xy_rdma(2, h, x_recv.at[h], y2_recv.at[h], y_dev)

        drain_state = {"n": 0}

        def out_copy(col_q, h, slot):
            return pltpu.make_async_copy(
                stage_ref.at[slot],
                out_ref.at[:, pl.ds(col_q * quarter + h * halfq, halfq)],
                stage_sems.at[slot],
            )

        pending = [None, None]

        def drain(value, col_q, h):
            slot = drain_state["n"] % 2
            drain_state["n"] += 1
            if pending[slot] is not None:
                out_copy(*pending[slot], slot).wait()
            stage_ref[slot] = value.astype(jnp.float32)
            out_copy(col_q, h, slot).start()
            pending[slot] = (col_q, h)

        for s in range(N_S):
            for h in range(N_H):
                recv_and_acc(s, h)
                if s + 1 < N_S:
                    start_z_sends(s + 1, h)
                else:
                    x_ex(h).start()
                    y_ex1(h).start()

        for h in range(N_H):
            drain(partial_ref[rows(my_z), hcols(h)], q, h)

        for h in range(N_H):
            x_ex(h).wait()
            y_ex2(h).start()
            drain(x_recv[h], q ^ 2, h)

        for h in range(N_H):
            y_ex1(h).wait()
            drain(y1_recv[h], q ^ 1, h)

        for h in range(N_H):
            y_ex2(h).wait()
            drain(y2_recv[h], q ^ 3, h)

        for slot in range(2):
            if pending[slot] is not None:
                out_copy(*pending[slot], slot).wait()

        for s in range(N_S):
            up_c = my_z + (N_Z - 1) - s
            dn_c = my_z - (N_Z - 1) + s
            for h in range(N_H):
                @pl.when(up_send_ok(s))
                def _():
                    z_rdma(0, s, h, up_c, my_z).wait_send()

                @pl.when(dn_send_ok(s))
                def _():
                    z_rdma(1, s, h, dn_c, my_z).wait_send()

    return pl.pallas_call(
        body,
        out_shape=jax.ShapeDtypeStruct((chunk, f), jnp.float32),
        in_specs=[
            pl.BlockSpec(memory_space=pltpu.VMEM),
            pl.BlockSpec(memory_space=pltpu.VMEM),
        ],
        out_specs=pl.BlockSpec(memory_space=pl.ANY),
        scratch_shapes=[
            pltpu.VMEM((m, quarter), jnp.bfloat16),
            pltpu.VMEM((N_S, N_H, chunk, halfq), jnp.bfloat16),
            pltpu.VMEM((N_S, N_H, chunk, halfq), jnp.bfloat16),
            pltpu.VMEM((N_H, chunk, halfq), jnp.bfloat16),
            pltpu.VMEM((N_H, chunk, halfq), jnp.bfloat16),
            pltpu.VMEM((N_H, chunk, halfq), jnp.bfloat16),
            pltpu.VMEM((2, chunk, halfq), jnp.float32),
            pltpu.SemaphoreType.DMA((2, N_S, N_H)),
            pltpu.SemaphoreType.DMA((2, N_S, N_H)),
            pltpu.SemaphoreType.DMA((3, N_H)),
            pltpu.SemaphoreType.DMA((3, N_H)),
            pltpu.SemaphoreType.DMA((2,)),
        ],
        compiler_params=pltpu.CompilerParams(
            collective_id=0,
            vmem_limit_bytes=64 * 1024 * 1024,
        ),
    )(x_bf, dy_bf)


# device time: 120431 ns/iter; 1.5513x vs baseline; 1.1179x over previous
import jax
import jax.numpy as jnp
from jax import lax
from jax.experimental import pallas as pl
from jax.experimental.pallas import tpu as pltpu

N_Z = 4
N_H = 4
N_S = N_Z - 1
N_KT = 8


def kernel(x, dy):
    k, m = x.shape
    _, f = dy.shape
    chunk = m // N_Z
    quarter = f // 4
    halfq = quarter // N_H
    tk = k // N_KT

    def body(x_hbm, dy_hbm, out_ref, x_bf, dy_bf, partial_ref, up_recv,
             dn_recv, x_recv, y1_recv, y2_recv, stage_ref, xs_ref, ds_ref,
             z_send_sems, z_recv_sems, xy_send_sems, xy_recv_sems,
             stage_sems, in_sems):
        my_x = lax.axis_index("x")
        my_y = lax.axis_index("y")
        my_z = lax.axis_index("z")
        q = 2 * my_x + my_y
        has_up = my_z < N_Z - 1
        has_dn = my_z > 0

        def in_copies(kt, slot):
            x_cp = pltpu.make_async_copy(
                x_hbm.at[pl.ds(kt * tk, tk), :],
                xs_ref.at[slot],
                in_sems.at[0, slot],
            )
            d_cp = pltpu.make_async_copy(
                dy_hbm.at[pl.ds(kt * tk, tk), pl.ds(q * quarter, quarter)],
                ds_ref.at[slot],
                in_sems.at[1, slot],
            )
            return x_cp, d_cp

        for kt in (0, 1):
            for cp in in_copies(kt, kt):
                cp.start()

        def up_send_ok(s):
            return jnp.logical_and(has_up, s >= my_z)

        def dn_send_ok(s):
            return jnp.logical_and(has_dn, s >= (N_Z - 1) - my_z)

        def rows(c):
            return pl.ds(c * chunk, chunk)

        def hcols(h):
            return pl.ds(h * halfq, halfq)

        def z_rdma(dir_i, s, h, c, target_z):
            recv = up_recv if dir_i == 0 else dn_recv
            return pltpu.make_async_remote_copy(
                src_ref=partial_ref.at[rows(c), hcols(h)],
                dst_ref=recv.at[s, h],
                send_sem=z_send_sems.at[dir_i, s, h],
                recv_sem=z_recv_sems.at[dir_i, s, h],
                device_id=(my_x, my_y, target_z),
                device_id_type=pl.DeviceIdType.MESH,
            )

        def start_z_sends(s, h):
            up_c = my_z + (N_Z - 1) - s
            dn_c = my_z - (N_Z - 1) + s

            @pl.when(up_send_ok(s))
            def _():
                z_rdma(0, s, h, up_c, my_z + 1).start()

            @pl.when(dn_send_ok(s))
            def _():
                z_rdma(1, s, h, dn_c, my_z - 1).start()

        def recv_and_acc(s, h):
            @pl.when(jnp.logical_and(has_dn, s >= my_z - 1))
            def _():
                z_rdma(0, s, h, 0, my_z).wait_recv()
                c = my_z + (N_Z - 2) - s
                partial_ref[rows(c), hcols(h)] = (
                    partial_ref[rows(c), hcols(h)] + up_recv[s, h]
                )

            @pl.when(jnp.logical_and(has_up, s >= (N_Z - 2) - my_z))
            def _():
                z_rdma(1, s, h, 0, my_z).wait_recv()
                c = my_z - (N_Z - 2) + s
                partial_ref[rows(c), hcols(h)] = (
                    partial_ref[rows(c), hcols(h)] + dn_recv[s, h]
                )

        barrier_sem = pltpu.get_barrier_semaphore()

        @pl.when(has_up)
        def _():
            pl.semaphore_signal(
                barrier_sem, inc=1, device_id=(my_x, my_y, my_z + 1),
                device_id_type=pl.DeviceIdType.MESH)

        @pl.when(has_dn)
        def _():
            pl.semaphore_signal(
                barrier_sem, inc=1, device_id=(my_x, my_y, my_z - 1),
                device_id_type=pl.DeviceIdType.MESH)

        pl.semaphore_signal(
            barrier_sem, inc=1, device_id=(1 - my_x, my_y, my_z),
            device_id_type=pl.DeviceIdType.MESH)
        pl.semaphore_signal(
            barrier_sem, inc=1, device_id=(my_x, 1 - my_y, my_z),
            device_id_type=pl.DeviceIdType.MESH)
        n_nbrs = 2 + has_up.astype(jnp.int32) + has_dn.astype(jnp.int32)
        pl.semaphore_wait(barrier_sem, n_nbrs)

        for kt in range(N_KT):
            slot = kt % 2
            for cp in in_copies(kt, slot):
                cp.wait()
            x_bf[pl.ds(kt * tk, tk), :] = xs_ref[slot].astype(jnp.bfloat16)
            dy_bf[pl.ds(kt * tk, tk), :] = ds_ref[slot].astype(jnp.bfloat16)
            if kt + 2 < N_KT:
                for cp in in_copies(kt + 2, slot):
                    cp.start()

        def gemm_block(r):
            partial_ref[r * chunk:(r + 1) * chunk, :] = lax.dot_general(
                x_bf[:, r * chunk:(r + 1) * chunk], dy_bf[...],
                dimension_numbers=(((0,), (0,)), ((), ())),
                preferred_element_type=jnp.float32,
            ).astype(jnp.bfloat16)

        gemm_block(N_Z - 1)
        gemm_block(0)
        for h in range(N_H):
            start_z_sends(0, h)
        gemm_block(2)
        gemm_block(1)

        own_src = [partial_ref.at[rows(my_z), hcols(h)] for h in range(N_H)]

        def xy_rdma(ex_i, h, src, dst, dev):
            return pltpu.make_async_remote_copy(
                src_ref=src, dst_ref=dst,
                send_sem=xy_send_sems.at[ex_i, h],
                recv_sem=xy_recv_sems.at[ex_i, h],
                device_id=dev, device_id_type=pl.DeviceIdType.MESH,
            )

        x_dev = (1 - my_x, my_y, my_z)
        y_dev = (my_x, 1 - my_y, my_z)

        def x_ex(h):
            return xy_rdma(0, h, own_src[h], x_recv.at[h], x_dev)

        def y_ex1(h):
            return xy_rdma(1, h, own_src[h], y1_recv.at[h], y_dev)

        def y_ex2(h):
            return xy_rdma(2, h, x_recv.at[h], y2_recv.at[h], y_dev)

        drain_state = {"n": 0}

        def out_copy(col_q, h, slot):
            return pltpu.make_async_copy(
                stage_ref.at[slot],
                out_ref.at[:, pl.ds(col_q * quarter + h * halfq, halfq)],
                stage_sems.at[slot],
            )

        pending = [None, None]

        def drain(value, col_q, h):
            slot = drain_state["n"] % 2
            drain_state["n"] += 1
            if pending[slot] is not None:
                out_copy(*pending[slot], slot).wait()
            stage_ref[slot] = value.astype(jnp.float32)
            out_copy(col_q, h, slot).start()
            pending[slot] = (col_q, h)

        for s in range(N_S):
            for h in range(N_H):
                recv_and_acc(s, h)
                if s + 1 < N_S:
                    start_z_sends(s + 1, h)
                else:
                    x_ex(h).start()
                    y_ex1(h).start()

        for h in range(N_H):
            drain(partial_ref[rows(my_z), hcols(h)], q, h)

        for h in range(N_H):
            x_ex(h).wait()
            y_ex2(h).start()
            drain(x_recv[h], q ^ 2, h)

        for h in range(N_H):
            y_ex1(h).wait()
            drain(y1_recv[h], q ^ 1, h)

        for h in range(N_H):
            y_ex2(h).wait()
            drain(y2_recv[h], q ^ 3, h)

        for slot in range(2):
            if pending[slot] is not None:
                out_copy(*pending[slot], slot).wait()

        for s in range(N_S):
            up_c = my_z + (N_Z - 1) - s
            dn_c = my_z - (N_Z - 1) + s
            for h in range(N_H):
                @pl.when(up_send_ok(s))
                def _():
                    z_rdma(0, s, h, up_c, my_z).wait_send()

                @pl.when(dn_send_ok(s))
                def _():
                    z_rdma(1, s, h, dn_c, my_z).wait_send()

    return pl.pallas_call(
        body,
        out_shape=jax.ShapeDtypeStruct((chunk, f), jnp.float32),
        in_specs=[
            pl.BlockSpec(memory_space=pl.ANY),
            pl.BlockSpec(memory_space=pl.ANY),
        ],
        out_specs=pl.BlockSpec(memory_space=pl.ANY),
        scratch_shapes=[
            pltpu.VMEM((k, m), jnp.bfloat16),
            pltpu.VMEM((k, quarter), jnp.bfloat16),
            pltpu.VMEM((m, quarter), jnp.bfloat16),
            pltpu.VMEM((N_S, N_H, chunk, halfq), jnp.bfloat16),
            pltpu.VMEM((N_S, N_H, chunk, halfq), jnp.bfloat16),
            pltpu.VMEM((N_H, chunk, halfq), jnp.bfloat16),
            pltpu.VMEM((N_H, chunk, halfq), jnp.bfloat16),
            pltpu.VMEM((N_H, chunk, halfq), jnp.bfloat16),
            pltpu.VMEM((2, chunk, halfq), jnp.float32),
            pltpu.VMEM((2, k // N_KT, m), jnp.float32),
            pltpu.VMEM((2, k // N_KT, quarter), jnp.float32),
            pltpu.SemaphoreType.DMA((2, N_S, N_H)),
            pltpu.SemaphoreType.DMA((2, N_S, N_H)),
            pltpu.SemaphoreType.DMA((3, N_H)),
            pltpu.SemaphoreType.DMA((3, N_H)),
            pltpu.SemaphoreType.DMA((2,)),
            pltpu.SemaphoreType.DMA((2, 2)),
        ],
        compiler_params=pltpu.CompilerParams(
            collective_id=0,
            vmem_limit_bytes=64 * 1024 * 1024,
        ),
    )(x, dy)
